# Initial kernel scaffold; baseline (speedup 1.0000x reference)
import jax
import jax.numpy as jnp
from jax.experimental import pallas as pl


def kernel(x, y, U):
    raise NotImplementedError("write your pallas kernel here")


# speedup vs baseline: 8.2986x; 8.2986x over previous
import math

import jax
import jax.numpy as jnp
from jax import lax
from jax.experimental import pallas as pl
from jax.experimental.pallas import tpu as pltpu

N = 4096
D = 64
L = 200
R = 8
M = 2 * N


def _w1_block(u0_ref, u1_ref, xt_ref, yt_ref, out_ref):
    xt = xt_ref[...]
    yt = yt_ref[...]
    u0 = u0_ref[...]
    u1 = u1_ref[...]

    xa = jnp.dot(u0, xt, preferred_element_type=jnp.float32)
    xb = jnp.dot(u1, xt, preferred_element_type=jnp.float32)
    ya = jnp.dot(u0, yt, preferred_element_type=jnp.float32)
    yb = jnp.dot(u1, yt, preferred_element_type=jnp.float32)

    two_pi_inv = 1.0 / (2.0 * math.pi)
    ax = (jnp.arctan2(-xb, -xa) + math.pi) * two_pi_inv
    ay = (jnp.arctan2(-yb, -ya) + math.pi) * two_pi_inv

    kx = pltpu.bitcast(ax, jnp.int32) | 1
    ky = pltpu.bitcast(ay, jnp.int32) & ~1
    keys = jnp.concatenate([kx, ky], axis=1)

    iota = lax.broadcasted_iota(jnp.int32, (R, M), 1)

    k = 2
    while k <= M:
        j = k >> 1
        while j >= 1:
            down = pltpu.roll(keys, M - j, 1)
            up = pltpu.roll(keys, j, 1)
            bitj0 = (iota & j) == 0
            partner = jnp.where(bitj0, down, up)
            asc = (iota & k) == 0
            keep_min = bitj0 == asc
            mn = jnp.minimum(keys, partner)
            mx = jnp.maximum(keys, partner)
            keys = jnp.where(keep_min, mn, mx)
            j >>= 1
        k <<= 1

    val = pltpu.bitcast(keys, jnp.float32)
    sgn = 2 * (keys & 1) - 1

    cdf = sgn
    sh = 1
    while sh < M:
        cdf = cdf + jnp.where(iota >= sh, pltpu.roll(cdf, sh, 1), 0)
        sh <<= 1

    nxt = jnp.where(iota == M - 1, 1.0, pltpu.roll(val, M - 1, 1))
    delta = nxt - val

    total = jnp.sum(delta, axis=1, keepdims=True)
    mincdf = jnp.min(cdf, axis=1, keepdims=True)

    lo = jnp.full((R, 1), -N, jnp.int32)
    hi = jnp.full((R, 1), N, jnp.int32)
    for _ in range(13):
        mid = lax.shift_right_arithmetic(lo + hi, 1)
        fmid = jnp.sum(jnp.where(cdf <= mid, delta, 0.0), axis=1, keepdims=True)
        ok = fmid >= 0.5
        hi = jnp.where(ok, mid, hi)
        lo = jnp.where(ok, lo, mid + 1)
    med = jnp.where(total >= 0.5, lo, mincdf)

    dev = jnp.abs(cdf - med).astype(jnp.float32)
    w = jnp.sum(delta * dev, axis=1) * (1.0 / N)
    out_ref[0, 0, :] = w


@jax.jit
def kernel(x, y, U):
    xt = x.T
    yt = y.T
    u0 = U[:, :, 0]
    u1 = U[:, :, 1]

    nb = L // R
    w = pl.pallas_call(
        _w1_block,
        grid=(nb,),
        in_specs=[
            pl.BlockSpec((R, D), lambda i: (i, 0)),
            pl.BlockSpec((R, D), lambda i: (i, 0)),
            pl.BlockSpec((D, N), lambda i: (0, 0)),
            pl.BlockSpec((D, N), lambda i: (0, 0)),
        ],
        out_specs=pl.BlockSpec((1, 1, R), lambda i: (i, 0, 0)),
        out_shape=jax.ShapeDtypeStruct((nb, 1, R), jnp.float32),
    )(u0, u1, xt, yt)
    return jnp.max(w)


# speedup vs baseline: 8.7579x; 1.0553x over previous
import math

import jax
import jax.numpy as jnp
from jax import lax
from jax.experimental import pallas as pl
from jax.experimental.pallas import tpu as pltpu

N = 4096
D = 64
L = 200
R = 8
M = 2 * N


def _w1_block(u0_ref, u1_ref, xt_ref, yt_ref, out_ref):
    xt = xt_ref[...]
    yt = yt_ref[...]
    u0 = u0_ref[...]
    u1 = u1_ref[...]

    xa = jnp.dot(u0, xt, preferred_element_type=jnp.float32)
    xb = jnp.dot(u1, xt, preferred_element_type=jnp.float32)
    ya = jnp.dot(u0, yt, preferred_element_type=jnp.float32)
    yb = jnp.dot(u1, yt, preferred_element_type=jnp.float32)

    two_pi_inv = 1.0 / (2.0 * math.pi)
    ax = (jnp.arctan2(-xb, -xa) + math.pi) * two_pi_inv
    ay = (jnp.arctan2(-yb, -ya) + math.pi) * two_pi_inv

    kx = pltpu.bitcast(ax, jnp.int32) | 1
    ky = pltpu.bitcast(ay, jnp.int32) & ~1
    keys = jnp.concatenate([kx, ky], axis=1)

    iota = lax.broadcasted_iota(jnp.int32, (R, M), 1)

    prev_flip = jnp.zeros((R, M), jnp.int32)
    k = 2
    while k <= M:
        flip = jnp.where((iota & k) == 0, 0, -1)
        keys = keys ^ (prev_flip ^ flip)
        prev_flip = flip
        j = k >> 1
        while j >= 1:
            down = pltpu.roll(keys, M - j, 1)
            up = pltpu.roll(keys, j, 1)
            bitj0 = (iota & j) == 0
            mn = jnp.minimum(keys, down)
            mx = jnp.maximum(keys, up)
            keys = jnp.where(bitj0, mn, mx)
            j >>= 1
        k <<= 1
    keys = keys ^ prev_flip

    val = pltpu.bitcast(keys, jnp.float32)
    sgn = 2 * (keys & 1) - 1

    cdf = sgn
    sh = 1
    while sh < M:
        cdf = cdf + jnp.where(iota >= sh, pltpu.roll(cdf, sh, 1), 0)
        sh <<= 1

    nxt = jnp.where(iota == M - 1, 1.0, pltpu.roll(val, M - 1, 1))
    delta = nxt - val

    total = jnp.sum(delta, axis=1, keepdims=True)
    mincdf = jnp.min(cdf, axis=1, keepdims=True)

    lo = jnp.full((R, 1), -N, jnp.int32)
    hi = jnp.full((R, 1), N, jnp.int32)
    for _ in range(13):
        mid = lax.shift_right_arithmetic(lo + hi, 1)
        fmid = jnp.sum(jnp.where(cdf <= mid, delta, 0.0), axis=1, keepdims=True)
        ok = fmid >= 0.5
        hi = jnp.where(ok, mid, hi)
        lo = jnp.where(ok, lo, mid + 1)
    med = jnp.where(total >= 0.5, lo, mincdf)

    dev = jnp.abs(cdf - med).astype(jnp.float32)
    w = jnp.sum(delta * dev, axis=1) * (1.0 / N)
    out_ref[0, 0, :] = w


@jax.jit
def kernel(x, y, U):
    xt = x.T
    yt = y.T
    u0 = U[:, :, 0]
    u1 = U[:, :, 1]

    nb = L // R
    w = pl.pallas_call(
        _w1_block,
        grid=(nb,),
        in_specs=[
            pl.BlockSpec((R, D), lambda i: (i, 0)),
            pl.BlockSpec((R, D), lambda i: (i, 0)),
            pl.BlockSpec((D, N), lambda i: (0, 0)),
            pl.BlockSpec((D, N), lambda i: (0, 0)),
        ],
        out_specs=pl.BlockSpec((1, 1, R), lambda i: (i, 0, 0)),
        out_shape=jax.ShapeDtypeStruct((nb, 1, R), jnp.float32),
    )(u0, u1, xt, yt)
    return jnp.max(w)
